# baseline (speedup 1.0000x reference)
import jax
import jax.numpy as jnp
from jax.experimental import pallas as pl
from jax.experimental.pallas import tpu as pltpu

NUM_EXPERTS = 8
NUM_GATES = 4
B, H, W, C = 4, 32, 32, 128
PIX = H * W
NPIX = B * PIX
EALL = NUM_EXPERTS * C
K9 = 9 * C
F32 = jnp.float32
BF16 = jnp.bfloat16


def _shift_rows(v, s):
    if s > 0:
        return jnp.concatenate([v[s:], jnp.zeros((s, C), v.dtype)], axis=0)
    if s < 0:
        return jnp.concatenate([jnp.zeros((-s, C), v.dtype), v[:NPIX + s]], axis=0)
    return v


def _moe_kernel(x_ref, g1_ref, g2_ref, g3_ref, g4_ref, wt_ref, bc_ref, wp_ref,
                bp_ref, y1_ref, y2_ref, y3_ref, y4_ref, loss_ref):
    xv = x_ref[...]

    lane_tok = jax.lax.broadcasted_iota(jnp.int32, (NUM_GATES, NPIX), 1) >> 10
    row4 = jax.lax.broadcasted_iota(jnp.int32, (NUM_GATES, NPIX), 0)
    rowsel = (lane_tok == row4).astype(F32)
    x0 = jnp.dot(rowsel, xv, preferred_element_type=F32,
                 precision=jax.lax.Precision.HIGHEST) * (1.0 / PIX)

    iota_c = jax.lax.broadcasted_iota(jnp.int32, (NUM_EXPERTS, NUM_GATES), 0)
    coeff_cols = []
    usage_cols = []
    ones_tok = jnp.ones((NUM_GATES, 1), F32)
    for g_ref in (g1_ref, g2_ref, g3_ref, g4_ref):
        lgt = jax.lax.dot_general(g_ref[...], x0, (((0,), (1,)), ((), ())),
                                  preferred_element_type=F32,
                                  precision=jax.lax.Precision.HIGHEST)
        lgt = lgt - jnp.max(lgt, axis=0, keepdims=True)
        el = jnp.exp(lgt)
        p = el / jnp.sum(el, axis=0, keepdims=True)
        usage_cols.append(jnp.dot(p, ones_tok, preferred_element_type=F32,
                                  precision=jax.lax.Precision.HIGHEST))
        m0 = jnp.max(p, axis=0, keepdims=True)
        i0 = jnp.min(jnp.where(p == m0, iota_c, NUM_EXPERTS), axis=0, keepdims=True)
        pm = jnp.where(iota_c == i0, -jnp.inf, p)
        m1 = jnp.max(pm, axis=0, keepdims=True)
        i1 = jnp.min(jnp.where(pm == m1, iota_c, NUM_EXPERTS), axis=0, keepdims=True)
        t = jnp.exp(m1 - m0)
        w0 = 1.0 / (1.0 + t)
        w1 = 1.0 - w0
        coeff_cols.append(w0 * (iota_c == i0).astype(F32)
                          + w1 * (iota_c == i1).astype(F32))

    xb = xv.astype(BF16)
    prow = jax.lax.broadcasted_iota(jnp.int32, (NPIX, 1), 0)
    pcol = prow & (W - 1)
    py = (prow >> 5) & (H - 1)
    taps = []
    for t in range(9):
        oy, ox = t // 3 - 1, t % 3 - 1
        sx = _shift_rows(xb, oy * W + ox)
        if ox == 1:
            sx = jnp.where(pcol == W - 1, BF16(0), sx)
        elif ox == -1:
            sx = jnp.where(pcol == 0, BF16(0), sx)
        if oy == 1:
            sx = jnp.where(py == H - 1, BF16(0), sx)
        elif oy == -1:
            sx = jnp.where(py == 0, BF16(0), sx)
        taps.append(sx)
    x9 = jnp.concatenate(taps, axis=1)
    u = jnp.dot(x9, wt_ref[...], preferred_element_type=F32)
    ub = (u + bc_ref[...]).astype(BF16)

    r8 = jax.lax.broadcasted_iota(jnp.int32, (EALL, NUM_EXPERTS), 0) >> 7
    c8 = jax.lax.broadcasted_iota(jnp.int32, (EALL, NUM_EXPERTS), 1)
    sel = (r8 == c8).astype(F32)
    sq = jnp.dot(ub * ub, sel.astype(BF16), preferred_element_type=F32)
    f = sq / ((1.0 + sq) * (jnp.sqrt(sq) + 1e-8))
    fb = f.astype(BF16)
    s_all = jnp.concatenate(
        [ub[:, e * C:(e + 1) * C] * fb[:, e:e + 1] for e in range(NUM_EXPERTS)],
        axis=1)

    wp = wp_ref[...]
    y_refs = (y1_ref, y2_ref, y3_ref, y4_ref)
    for g in range(NUM_GATES):
        pieces = []
        for tok in range(B):
            cc = coeff_cols[g][:, tok:tok + 1]
            c_col = jnp.dot(sel, cc, preferred_element_type=F32)
            wpg = wp * c_col.astype(BF16)
            s_tok = s_all[tok * PIX:(tok + 1) * PIX, :]
            yg = jnp.dot(s_tok, wpg, preferred_element_type=F32)
            bias = jax.lax.dot_general(cc, bp_ref[...], (((0,), (0,)), ((), ())),
                                       preferred_element_type=F32)
            pieces.append(yg + bias)
        y_refs[g][...] = jnp.concatenate(pieces, axis=0)

    usage = jnp.concatenate(usage_cols, axis=1)
    mean = jnp.mean(usage, axis=0, keepdims=True)
    var = jnp.sum((usage - mean) ** 2, axis=0, keepdims=True) / (NUM_EXPERTS - 1)
    cv = var / (mean * mean + 1e-10)
    total = jnp.sum(cv, axis=1, keepdims=True)
    loss_ref[...] = jnp.broadcast_to(total, (1, NUM_EXPERTS))


def kernel(x, gate1, gate2, gate3, gate4, Wc, bc, Wp, bp):
    xt = jnp.transpose(x, (0, 2, 3, 1)).reshape(NPIX, C)
    wt = jnp.transpose(Wc.astype(BF16), (3, 4, 2, 0, 1)).reshape(K9, EALL)
    bc_all = bc.reshape(1, EALL)
    wps = jnp.transpose(Wp[:, :, :, 0, 0].astype(BF16), (0, 2, 1)).reshape(EALL, C)

    gspec = pl.BlockSpec((C, NUM_EXPERTS), lambda: (0, 0))
    yspec = pl.BlockSpec((NPIX, C), lambda: (0, 0))
    yshape = jax.ShapeDtypeStruct((NPIX, C), F32)
    outs = pl.pallas_call(
        _moe_kernel,
        in_specs=[
            pl.BlockSpec((NPIX, C), lambda: (0, 0)),
            gspec, gspec, gspec, gspec,
            pl.BlockSpec((K9, EALL), lambda: (0, 0)),
            pl.BlockSpec((1, EALL), lambda: (0, 0)),
            pl.BlockSpec((EALL, C), lambda: (0, 0)),
            pl.BlockSpec((NUM_EXPERTS, C), lambda: (0, 0)),
        ],
        out_specs=[
            yspec, yspec, yspec, yspec,
            pl.BlockSpec((1, NUM_EXPERTS), lambda: (0, 0)),
        ],
        out_shape=[
            yshape, yshape, yshape, yshape,
            jax.ShapeDtypeStruct((1, NUM_EXPERTS), F32),
        ],
    )(xt, gate1, gate2, gate3, gate4, wt, bc_all, wps, bp)

    ys = [o.reshape(B, H, W, C).transpose(0, 3, 1, 2) for o in outs[:4]]
    l = outs[4][0, 0].reshape(())
    return (ys[0], ys[1], ys[2], ys[3], l)


# speedup vs baseline: 1.1049x; 1.1049x over previous
import jax
import jax.numpy as jnp
from jax.experimental import pallas as pl
from jax.experimental.pallas import tpu as pltpu

NUM_EXPERTS = 8
NUM_GATES = 4
B, H, W, C = 4, 32, 32, 128
PIX = H * W
NPIX = B * PIX
EALL = NUM_EXPERTS * C
K9 = 9 * C
F32 = jnp.float32
BF16 = jnp.bfloat16


def _shift_rows(v, s):
    if s > 0:
        return jnp.concatenate([v[s:], jnp.zeros((s, C), v.dtype)], axis=0)
    if s < 0:
        return jnp.concatenate([jnp.zeros((-s, C), v.dtype), v[:NPIX + s]], axis=0)
    return v


def _moe_kernel(x_ref, g1_ref, g2_ref, g3_ref, g4_ref, wt_ref, bc_ref, wp_ref,
                bp_ref, y1_ref, y2_ref, y3_ref, y4_ref, loss_ref):
    xv = x_ref[...]

    lane_tok = jax.lax.broadcasted_iota(jnp.int32, (NUM_GATES, NPIX), 1) >> 10
    row4 = jax.lax.broadcasted_iota(jnp.int32, (NUM_GATES, NPIX), 0)
    rowsel = (lane_tok == row4).astype(F32)
    x0 = jnp.dot(rowsel, xv, preferred_element_type=F32,
                 precision=jax.lax.Precision.HIGHEST) * (1.0 / PIX)

    iota_c = jax.lax.broadcasted_iota(jnp.int32, (NUM_EXPERTS, NUM_GATES), 0)
    coeff_cols = []
    usage_cols = []
    ones_tok = jnp.ones((NUM_GATES, 1), F32)
    for g_ref in (g1_ref, g2_ref, g3_ref, g4_ref):
        lgt = jax.lax.dot_general(g_ref[...], x0, (((0,), (1,)), ((), ())),
                                  preferred_element_type=F32,
                                  precision=jax.lax.Precision.HIGHEST)
        lgt = lgt - jnp.max(lgt, axis=0, keepdims=True)
        el = jnp.exp(lgt)
        p = el / jnp.sum(el, axis=0, keepdims=True)
        usage_cols.append(jnp.dot(p, ones_tok, preferred_element_type=F32,
                                  precision=jax.lax.Precision.HIGHEST))
        m0 = jnp.max(p, axis=0, keepdims=True)
        i0 = jnp.min(jnp.where(p == m0, iota_c, NUM_EXPERTS), axis=0, keepdims=True)
        pm = jnp.where(iota_c == i0, -jnp.inf, p)
        m1 = jnp.max(pm, axis=0, keepdims=True)
        i1 = jnp.min(jnp.where(pm == m1, iota_c, NUM_EXPERTS), axis=0, keepdims=True)
        t = jnp.exp(m1 - m0)
        w0 = 1.0 / (1.0 + t)
        w1 = 1.0 - w0
        coeff_cols.append(w0 * (iota_c == i0).astype(F32)
                          + w1 * (iota_c == i1).astype(F32))

    xb = xv.astype(BF16)
    prow = jax.lax.broadcasted_iota(jnp.int32, (NPIX, 1), 0)
    pcol = prow & (W - 1)
    py = (prow >> 5) & (H - 1)
    taps = []
    for t in range(9):
        oy, ox = t // 3 - 1, t % 3 - 1
        sx = _shift_rows(xb, oy * W + ox)
        if ox == 1:
            sx = jnp.where(pcol == W - 1, BF16(0), sx)
        elif ox == -1:
            sx = jnp.where(pcol == 0, BF16(0), sx)
        if oy == 1:
            sx = jnp.where(py == H - 1, BF16(0), sx)
        elif oy == -1:
            sx = jnp.where(py == 0, BF16(0), sx)
        taps.append(sx)
    x9 = jnp.concatenate(taps, axis=1)
    u = jnp.dot(x9, wt_ref[...], preferred_element_type=F32)
    ub = (u + bc_ref[...]).astype(BF16)

    r8 = jax.lax.broadcasted_iota(jnp.int32, (EALL, NUM_EXPERTS), 0) >> 7
    c8 = jax.lax.broadcasted_iota(jnp.int32, (EALL, NUM_EXPERTS), 1)
    sel = (r8 == c8).astype(F32)
    sq = jnp.dot(ub * ub, sel.astype(BF16), preferred_element_type=F32)
    f = sq / ((1.0 + sq) * (jnp.sqrt(sq) + 1e-8))
    fb = f.astype(BF16)
    s_all = jnp.concatenate(
        [ub[:, e * C:(e + 1) * C] * fb[:, e:e + 1] for e in range(NUM_EXPERTS)],
        axis=1)

    wp = wp_ref[...]
    y_refs = (y1_ref, y2_ref, y3_ref, y4_ref)
    gate_pieces = [[] for _ in range(NUM_GATES)]
    for tok in range(B):
        wpg4 = jnp.concatenate(
            [wp * jnp.dot(sel, coeff_cols[g][:, tok:tok + 1],
                          preferred_element_type=F32).astype(BF16)
             for g in range(NUM_GATES)], axis=1)
        s_tok = s_all[tok * PIX:(tok + 1) * PIX, :]
        y4 = jnp.dot(s_tok, wpg4, preferred_element_type=F32)
        for g in range(NUM_GATES):
            cc = coeff_cols[g][:, tok:tok + 1]
            bias = jax.lax.dot_general(cc, bp_ref[...], (((0,), (0,)), ((), ())),
                                       preferred_element_type=F32)
            gate_pieces[g].append(y4[:, g * C:(g + 1) * C] + bias)
    for g in range(NUM_GATES):
        y_refs[g][...] = jnp.concatenate(gate_pieces[g], axis=0)

    usage = jnp.concatenate(usage_cols, axis=1)
    mean = jnp.mean(usage, axis=0, keepdims=True)
    var = jnp.sum((usage - mean) ** 2, axis=0, keepdims=True) / (NUM_EXPERTS - 1)
    cv = var / (mean * mean + 1e-10)
    total = jnp.sum(cv, axis=1, keepdims=True)
    loss_ref[...] = jnp.broadcast_to(total, (1, NUM_EXPERTS))


def kernel(x, gate1, gate2, gate3, gate4, Wc, bc, Wp, bp):
    xt = jnp.transpose(x, (0, 2, 3, 1)).reshape(NPIX, C)
    wt = jnp.transpose(Wc.astype(BF16), (3, 4, 2, 0, 1)).reshape(K9, EALL)
    bc_all = bc.reshape(1, EALL)
    wps = jnp.transpose(Wp[:, :, :, 0, 0].astype(BF16), (0, 2, 1)).reshape(EALL, C)

    gspec = pl.BlockSpec((C, NUM_EXPERTS), lambda: (0, 0))
    yspec = pl.BlockSpec((NPIX, C), lambda: (0, 0))
    yshape = jax.ShapeDtypeStruct((NPIX, C), F32)
    outs = pl.pallas_call(
        _moe_kernel,
        in_specs=[
            pl.BlockSpec((NPIX, C), lambda: (0, 0)),
            gspec, gspec, gspec, gspec,
            pl.BlockSpec((K9, EALL), lambda: (0, 0)),
            pl.BlockSpec((1, EALL), lambda: (0, 0)),
            pl.BlockSpec((EALL, C), lambda: (0, 0)),
            pl.BlockSpec((NUM_EXPERTS, C), lambda: (0, 0)),
        ],
        out_specs=[
            yspec, yspec, yspec, yspec,
            pl.BlockSpec((1, NUM_EXPERTS), lambda: (0, 0)),
        ],
        out_shape=[
            yshape, yshape, yshape, yshape,
            jax.ShapeDtypeStruct((1, NUM_EXPERTS), F32),
        ],
    )(xt, gate1, gate2, gate3, gate4, wt, bc_all, wps, bp)

    ys = [o.reshape(B, H, W, C).transpose(0, 3, 1, 2) for o in outs[:4]]
    l = outs[4][0, 0].reshape(())
    return (ys[0], ys[1], ys[2], ys[3], l)


# speedup vs baseline: 1.1376x; 1.0295x over previous
import jax
import jax.numpy as jnp
from jax.experimental import pallas as pl
from jax.experimental.pallas import tpu as pltpu

NUM_EXPERTS = 8
NUM_GATES = 4
B, H, W, C = 4, 32, 32, 128
PIX = H * W
NPIX = B * PIX
TSTEP = 2
SPIX = TSTEP * PIX
EALL = NUM_EXPERTS * C
K9 = 9 * C
F32 = jnp.float32
BF16 = jnp.bfloat16


def _shift_rows(v, s):
    n = v.shape[0]
    if s > 0:
        return jnp.concatenate([v[s:], jnp.zeros((s, C), v.dtype)], axis=0)
    if s < 0:
        return jnp.concatenate([jnp.zeros((-s, C), v.dtype), v[:n + s]], axis=0)
    return v


def _moe_kernel(x_ref, g1_ref, g2_ref, g3_ref, g4_ref, wt_ref, bc_ref, wp_ref,
                bp_ref, y1_ref, y2_ref, y3_ref, y4_ref, loss_ref, usage_ref):
    step = pl.program_id(0)
    xv = x_ref[...]

    lane_tok = jax.lax.broadcasted_iota(jnp.int32, (TSTEP, SPIX), 1) >> 10
    row4 = jax.lax.broadcasted_iota(jnp.int32, (TSTEP, SPIX), 0)
    rowsel = (lane_tok == row4).astype(F32)
    x0 = jnp.dot(rowsel, xv, preferred_element_type=F32,
                 precision=jax.lax.Precision.HIGHEST) * (1.0 / PIX)

    iota_c = jax.lax.broadcasted_iota(jnp.int32, (NUM_EXPERTS, TSTEP), 0)
    coeff_cols = []
    usage_cols = []
    ones_tok = jnp.ones((TSTEP, 1), F32)
    for g_ref in (g1_ref, g2_ref, g3_ref, g4_ref):
        lgt = jax.lax.dot_general(g_ref[...], x0, (((0,), (1,)), ((), ())),
                                  preferred_element_type=F32,
                                  precision=jax.lax.Precision.HIGHEST)
        lgt = lgt - jnp.max(lgt, axis=0, keepdims=True)
        el = jnp.exp(lgt)
        p = el / jnp.sum(el, axis=0, keepdims=True)
        usage_cols.append(jnp.dot(p, ones_tok, preferred_element_type=F32,
                                  precision=jax.lax.Precision.HIGHEST))
        m0 = jnp.max(p, axis=0, keepdims=True)
        i0 = jnp.min(jnp.where(p == m0, iota_c, NUM_EXPERTS), axis=0, keepdims=True)
        pm = jnp.where(iota_c == i0, -jnp.inf, p)
        m1 = jnp.max(pm, axis=0, keepdims=True)
        i1 = jnp.min(jnp.where(pm == m1, iota_c, NUM_EXPERTS), axis=0, keepdims=True)
        t = jnp.exp(m1 - m0)
        w0 = 1.0 / (1.0 + t)
        w1 = 1.0 - w0
        coeff_cols.append(w0 * (iota_c == i0).astype(F32)
                          + w1 * (iota_c == i1).astype(F32))

    xb = xv.astype(BF16)
    prow = jax.lax.broadcasted_iota(jnp.int32, (SPIX, 1), 0)
    pcol = prow & (W - 1)
    py = (prow >> 5) & (H - 1)
    taps = []
    for t in range(9):
        oy, ox = t // 3 - 1, t % 3 - 1
        sx = _shift_rows(xb, oy * W + ox)
        if ox == 1:
            sx = jnp.where(pcol == W - 1, BF16(0), sx)
        elif ox == -1:
            sx = jnp.where(pcol == 0, BF16(0), sx)
        if oy == 1:
            sx = jnp.where(py == H - 1, BF16(0), sx)
        elif oy == -1:
            sx = jnp.where(py == 0, BF16(0), sx)
        taps.append(sx)
    x9 = jnp.concatenate(taps, axis=1)
    u = jnp.dot(x9, wt_ref[...], preferred_element_type=F32)
    ub = (u + bc_ref[...]).astype(BF16)

    r8 = jax.lax.broadcasted_iota(jnp.int32, (EALL, NUM_EXPERTS), 0) >> 7
    c8 = jax.lax.broadcasted_iota(jnp.int32, (EALL, NUM_EXPERTS), 1)
    sel = (r8 == c8).astype(F32)
    sq = jnp.dot(ub * ub, sel.astype(BF16), preferred_element_type=F32)
    f = sq / ((1.0 + sq) * (jnp.sqrt(sq) + 1e-8))
    fb = f.astype(BF16)
    s_all = jnp.concatenate(
        [ub[:, e * C:(e + 1) * C] * fb[:, e:e + 1] for e in range(NUM_EXPERTS)],
        axis=1)

    wp = wp_ref[...]
    y_refs = (y1_ref, y2_ref, y3_ref, y4_ref)
    gate_pieces = [[] for _ in range(NUM_GATES)]
    for tok in range(TSTEP):
        wpg4 = jnp.concatenate(
            [wp * jnp.dot(sel, coeff_cols[g][:, tok:tok + 1],
                          preferred_element_type=F32).astype(BF16)
             for g in range(NUM_GATES)], axis=1)
        s_tok = s_all[tok * PIX:(tok + 1) * PIX, :]
        y4 = jnp.dot(s_tok, wpg4, preferred_element_type=F32)
        for g in range(NUM_GATES):
            cc = coeff_cols[g][:, tok:tok + 1]
            bias = jax.lax.dot_general(cc, bp_ref[...], (((0,), (0,)), ((), ())),
                                       preferred_element_type=F32)
            gate_pieces[g].append(y4[:, g * C:(g + 1) * C] + bias)
    for g in range(NUM_GATES):
        y_refs[g][...] = jnp.concatenate(gate_pieces[g], axis=0)

    step_usage = jnp.concatenate(usage_cols, axis=1)

    @pl.when(step == 0)
    def _():
        usage_ref[...] = step_usage

    @pl.when(step > 0)
    def _():
        usage_ref[...] += step_usage

    @pl.when(step == B // TSTEP - 1)
    def _():
        usage = usage_ref[...]
        mean = jnp.mean(usage, axis=0, keepdims=True)
        var = jnp.sum((usage - mean) ** 2, axis=0,
                      keepdims=True) / (NUM_EXPERTS - 1)
        cv = var / (mean * mean + 1e-10)
        total = jnp.sum(cv, axis=1, keepdims=True)
        loss_ref[...] = jnp.broadcast_to(total, (1, NUM_EXPERTS))


def kernel(x, gate1, gate2, gate3, gate4, Wc, bc, Wp, bp):
    xt = jnp.transpose(x, (0, 2, 3, 1)).reshape(NPIX, C)
    wt = jnp.transpose(Wc.astype(BF16), (3, 4, 2, 0, 1)).reshape(K9, EALL)
    bc_all = bc.reshape(1, EALL)
    wps = jnp.transpose(Wp[:, :, :, 0, 0].astype(BF16), (0, 2, 1)).reshape(EALL, C)

    gspec = pl.BlockSpec((C, NUM_EXPERTS), lambda i: (0, 0))
    yspec = pl.BlockSpec((SPIX, C), lambda i: (i, 0))
    yshape = jax.ShapeDtypeStruct((NPIX, C), F32)
    outs = pl.pallas_call(
        _moe_kernel,
        grid=(B // TSTEP,),
        in_specs=[
            pl.BlockSpec((SPIX, C), lambda i: (i, 0)),
            gspec, gspec, gspec, gspec,
            pl.BlockSpec((K9, EALL), lambda i: (0, 0)),
            pl.BlockSpec((1, EALL), lambda i: (0, 0)),
            pl.BlockSpec((EALL, C), lambda i: (0, 0)),
            pl.BlockSpec((NUM_EXPERTS, C), lambda i: (0, 0)),
        ],
        out_specs=[
            yspec, yspec, yspec, yspec,
            pl.BlockSpec((1, NUM_EXPERTS), lambda i: (0, 0)),
        ],
        out_shape=[
            yshape, yshape, yshape, yshape,
            jax.ShapeDtypeStruct((1, NUM_EXPERTS), F32),
        ],
        scratch_shapes=[pltpu.VMEM((NUM_EXPERTS, NUM_GATES), F32)],
        compiler_params=pltpu.CompilerParams(
            dimension_semantics=("arbitrary",)),
    )(xt, gate1, gate2, gate3, gate4, wt, bc_all, wps, bp)

    ys = [o.reshape(B, H, W, C).transpose(0, 3, 1, 2) for o in outs[:4]]
    l = outs[4][0, 0].reshape(())
    return (ys[0], ys[1], ys[2], ys[3], l)
